# Initial kernel scaffold; baseline (speedup 1.0000x reference)
import functools

import numpy as np
import jax
import jax.numpy as jnp
from jax.experimental import pallas as pl
from jax.experimental.pallas import tpu as pltpu

_N = 20000
_G = 64
_TOPK = 10
_SCALE_CLAMP = float(np.log(1000.0 / 16))


def _sim_ota_body(anchors_t, deltas_t, gt, cls2, stride2, out_ref):
    ax0 = anchors_t[0:1, :]
    ay0 = anchors_t[1:2, :]
    ax1 = anchors_t[2:3, :]
    ay1 = anchors_t[3:4, :]
    d0 = deltas_t[0:1, :]
    d1 = deltas_t[1:2, :]
    d2 = deltas_t[2:3, :]
    d3 = deltas_t[3:4, :]

    widths = ax1 - ax0
    heights = ay1 - ay0
    ctr_x = ax0 + 0.5 * widths
    ctr_y = ay0 + 0.5 * heights
    dx = d0 / 10.0
    dy = d1 / 10.0
    dw = jnp.minimum(d2 / 5.0, _SCALE_CLAMP)
    dh = jnp.minimum(d3 / 5.0, _SCALE_CLAMP)
    pcx = dx * widths + ctr_x
    pcy = dy * heights + ctr_y
    pw = jnp.exp(dw) * widths
    ph = jnp.exp(dh) * heights
    px0 = pcx - 0.5 * pw
    py0 = pcy - 0.5 * ph
    px1 = pcx + 0.5 * pw
    py1 = pcy + 0.5 * ph

    x_shifts = (ax0 + ax1) / 2.0
    y_shifts = (ay0 + ay1) / 2.0

    g0 = gt[:, 0:1]
    g1 = gt[:, 1:2]
    g2 = gt[:, 2:3]
    g3 = gt[:, 3:4]

    cdist = 1.5 * stride2
    gt_cx = (g0 + g2) / 2.0
    gt_cy = (g1 + g3) / 2.0
    in_cx = jnp.abs(x_shifts - gt_cx) < cdist
    in_cy = jnp.abs(y_shifts - gt_cy) < cdist
    geom = in_cx & in_cy

    area_a = (g2 - g0) * (g3 - g1)
    area_p = (px1 - px0) * (py1 - py0)
    ltx = jnp.maximum(g0, px0)
    lty = jnp.maximum(g1, py0)
    rbx = jnp.minimum(g2, px1)
    rby = jnp.minimum(g3, py1)
    whx = jnp.clip(rbx - ltx, 0.0, None)
    why = jnp.clip(rby - lty, 0.0, None)
    inter = whx * why
    union = area_a + area_p - inter
    iou = inter / jnp.maximum(union, 1e-8)

    iou_loss = -jnp.log(iou + 1e-8)
    p = jax.nn.sigmoid(cls2)
    cls_loss = -jnp.log(p + 1e-12)
    cost = cls_loss + 3.0 * iou_loss + jnp.where(geom, 0.0, 1e6)

    col_ids = jax.lax.broadcasted_iota(jnp.int32, (_G, _N), 1)
    row_ids = jax.lax.broadcasted_iota(jnp.int32, (_G, _N), 0)

    iou_cand = jnp.where(geom, iou, 0.0)

    def _topk_iou(i, carry):
        s, work = carry
        m = jnp.max(work, axis=1, keepdims=True)
        amax = jnp.min(jnp.where(work == m, col_ids, _N), axis=1, keepdims=True)
        work = jnp.where(col_ids == amax, -jnp.inf, work)
        return s + m, work

    s0 = jnp.zeros((_G, 1), jnp.float32)
    s, _ = jax.lax.fori_loop(0, _TOPK, _topk_iou, (s0, iou_cand))
    dyn_k = jnp.maximum(s, 1.0).astype(jnp.int32)

    def _topk_cost(i, carry):
        matching, work = carry
        m = jnp.min(work, axis=1, keepdims=True)
        amin = jnp.min(jnp.where(work == m, col_ids, _N), axis=1, keepdims=True)
        hit = col_ids == amin
        sel = hit & (dyn_k > i) & geom
        matching = jnp.where(sel, 1.0, matching)
        work = jnp.where(hit, jnp.inf, work)
        return matching, work

    m0 = jnp.zeros((_G, _N), jnp.float32)
    matching, _ = jax.lax.fori_loop(0, _TOPK, _topk_cost, (m0, cost))

    nmatch = jnp.sum(matching, axis=0, keepdims=True)
    multi = nmatch > 1.0
    cmin = jnp.min(cost, axis=0, keepdims=True)
    best_gt = jnp.min(jnp.where(cost == cmin, row_ids, _G), axis=0, keepdims=True)
    onehot = row_ids == best_gt
    matching = jnp.where(multi & ~onehot, 0.0, matching)

    fg = jnp.sum(matching, axis=0, keepdims=True) > 0.0
    pred_ious = jnp.sum(jnp.where(matching > 0.0, iou, 0.0), axis=0, keepdims=True)

    area_b = (ax1 - ax0) * (ay1 - ay0)
    ltx2 = jnp.maximum(g0, ax0)
    lty2 = jnp.maximum(g1, ay0)
    rbx2 = jnp.minimum(g2, ax1)
    rby2 = jnp.minimum(g3, ay1)
    whx2 = jnp.clip(rbx2 - ltx2, 0.0, None)
    why2 = jnp.clip(rby2 - lty2, 0.0, None)
    inter2 = whx2 * why2
    union2 = area_a + area_b - inter2
    iou2 = inter2 / jnp.maximum(union2, 1e-8)
    max_anchor_iou = jnp.max(iou2, axis=0, keepdims=True)

    label = jnp.where(fg, 1.0,
                      jnp.where(max_anchor_iou >= 0.3, -1.0, 0.0))

    bx0 = jnp.sum(matching * g0, axis=0, keepdims=True)
    bx1 = jnp.sum(matching * g1, axis=0, keepdims=True)
    bx2 = jnp.sum(matching * g2, axis=0, keepdims=True)
    bx3 = jnp.sum(matching * g3, axis=0, keepdims=True)

    zero = jnp.zeros((1, _N), jnp.float32)
    out_ref[:, :] = jnp.concatenate(
        [label, pred_ious, bx0, bx1, bx2, bx3, zero, zero], axis=0)


@jax.jit
def kernel(anchors, pred_deltas, gt_boxes, cls_preds, expanded_strides, gt_classes):
    del gt_classes
    anchors_t = anchors.T
    deltas_t = pred_deltas.T
    cls2 = cls_preds.reshape(1, _N)
    stride2 = expanded_strides.reshape(1, _N)
    out = pl.pallas_call(
        _sim_ota_body,
        out_shape=jax.ShapeDtypeStruct((8, _N), jnp.float32),
    )(anchors_t, deltas_t, gt_boxes, cls2, stride2)
    return out[:6, :].T


# speedup vs baseline: 24.5205x; 24.5205x over previous
import functools

import numpy as np
import jax
import jax.numpy as jnp
from jax.experimental import pallas as pl
from jax.experimental.pallas import tpu as pltpu

_N = 20000
_G = 64
_TOPK = 10
_SCALE_CLAMP = float(np.log(1000.0 / 16))


def _sim_ota_body(anchors_t, deltas_t, gt, cls2, stride2, out_ref):
    ax0 = anchors_t[0:1, :]
    ay0 = anchors_t[1:2, :]
    ax1 = anchors_t[2:3, :]
    ay1 = anchors_t[3:4, :]
    d0 = deltas_t[0:1, :]
    d1 = deltas_t[1:2, :]
    d2 = deltas_t[2:3, :]
    d3 = deltas_t[3:4, :]

    widths = ax1 - ax0
    heights = ay1 - ay0
    ctr_x = ax0 + 0.5 * widths
    ctr_y = ay0 + 0.5 * heights
    dx = d0 / 10.0
    dy = d1 / 10.0
    dw = jnp.minimum(d2 / 5.0, _SCALE_CLAMP)
    dh = jnp.minimum(d3 / 5.0, _SCALE_CLAMP)
    pcx = dx * widths + ctr_x
    pcy = dy * heights + ctr_y
    pw = jnp.exp(dw) * widths
    ph = jnp.exp(dh) * heights
    px0 = pcx - 0.5 * pw
    py0 = pcy - 0.5 * ph
    px1 = pcx + 0.5 * pw
    py1 = pcy + 0.5 * ph

    x_shifts = (ax0 + ax1) / 2.0
    y_shifts = (ay0 + ay1) / 2.0

    g0 = gt[:, 0:1]
    g1 = gt[:, 1:2]
    g2 = gt[:, 2:3]
    g3 = gt[:, 3:4]

    cdist = 1.5 * stride2[...]
    gt_cx = (g0 + g2) / 2.0
    gt_cy = (g1 + g3) / 2.0
    in_cx = jnp.abs(x_shifts - gt_cx) < cdist
    in_cy = jnp.abs(y_shifts - gt_cy) < cdist
    geom = in_cx & in_cy

    area_a = (g2 - g0) * (g3 - g1)
    area_p = (px1 - px0) * (py1 - py0)
    ltx = jnp.maximum(g0, px0)
    lty = jnp.maximum(g1, py0)
    rbx = jnp.minimum(g2, px1)
    rby = jnp.minimum(g3, py1)
    whx = jnp.clip(rbx - ltx, 0.0, None)
    why = jnp.clip(rby - lty, 0.0, None)
    inter = whx * why
    union = area_a + area_p - inter
    iou = inter / jnp.maximum(union, 1e-8)

    iou_loss = -jnp.log(iou + 1e-8)
    p = jax.nn.sigmoid(cls2[...])
    cls_loss = -jnp.log(p + 1e-12)
    cost = cls_loss + 3.0 * iou_loss + jnp.where(geom, 0.0, 1e6)

    col_ids = jax.lax.broadcasted_iota(jnp.int32, (_G, _N), 1)
    row_ids = jax.lax.broadcasted_iota(jnp.int32, (_G, _N), 0)

    iou_cand = jnp.where(geom, iou, 0.0)

    def _topk_iou(i, carry):
        s, work = carry
        m = jnp.max(work, axis=1, keepdims=True)
        amax = jnp.min(jnp.where(work == m, col_ids, _N), axis=1, keepdims=True)
        work = jnp.where(col_ids == amax, -jnp.inf, work)
        return s + m, work

    s0 = jnp.zeros((_G, 1), jnp.float32)
    s, _ = jax.lax.fori_loop(0, _TOPK, _topk_iou, (s0, iou_cand))
    dyn_k = jnp.maximum(s, 1.0).astype(jnp.int32)

    def _topk_cost(i, carry):
        matching, work = carry
        m = jnp.min(work, axis=1, keepdims=True)
        amin = jnp.min(jnp.where(work == m, col_ids, _N), axis=1, keepdims=True)
        hit = col_ids == amin
        sel = hit & (dyn_k > i) & geom
        matching = jnp.where(sel, 1.0, matching)
        work = jnp.where(hit, jnp.inf, work)
        return matching, work

    m0 = jnp.zeros((_G, _N), jnp.float32)
    matching, _ = jax.lax.fori_loop(0, _TOPK, _topk_cost, (m0, cost))

    nmatch = jnp.sum(matching, axis=0, keepdims=True)
    multi = nmatch > 1.0
    cmin = jnp.min(cost, axis=0, keepdims=True)
    best_gt = jnp.min(jnp.where(cost == cmin, row_ids, _G), axis=0, keepdims=True)
    onehot = row_ids == best_gt
    matching = jnp.where(multi & ~onehot, 0.0, matching)

    fg = jnp.sum(matching, axis=0, keepdims=True) > 0.0
    pred_ious = jnp.sum(jnp.where(matching > 0.0, iou, 0.0), axis=0, keepdims=True)

    area_b = (ax1 - ax0) * (ay1 - ay0)
    ltx2 = jnp.maximum(g0, ax0)
    lty2 = jnp.maximum(g1, ay0)
    rbx2 = jnp.minimum(g2, ax1)
    rby2 = jnp.minimum(g3, ay1)
    whx2 = jnp.clip(rbx2 - ltx2, 0.0, None)
    why2 = jnp.clip(rby2 - lty2, 0.0, None)
    inter2 = whx2 * why2
    union2 = area_a + area_b - inter2
    iou2 = inter2 / jnp.maximum(union2, 1e-8)
    max_anchor_iou = jnp.max(iou2, axis=0, keepdims=True)

    label = jnp.where(fg, 1.0,
                      jnp.where(max_anchor_iou >= 0.3, -1.0, 0.0))

    bx0 = jnp.sum(matching * g0, axis=0, keepdims=True)
    bx1 = jnp.sum(matching * g1, axis=0, keepdims=True)
    bx2 = jnp.sum(matching * g2, axis=0, keepdims=True)
    bx3 = jnp.sum(matching * g3, axis=0, keepdims=True)

    zero = jnp.zeros((1, _N), jnp.float32)
    out_ref[:, :] = jnp.concatenate(
        [label, pred_ious, bx0, bx1, bx2, bx3, zero, zero], axis=0)


@jax.jit
def kernel(anchors, pred_deltas, gt_boxes, cls_preds, expanded_strides, gt_classes):
    del gt_classes
    anchors_t = anchors.T
    deltas_t = pred_deltas.T
    cls2 = cls_preds.reshape(1, _N)
    stride2 = expanded_strides.reshape(1, _N)
    out = pl.pallas_call(
        _sim_ota_body,
        out_shape=jax.ShapeDtypeStruct((8, _N), jnp.float32),
    )(anchors_t, deltas_t, gt_boxes, cls2, stride2)
    return out[:6, :].T


# speedup vs baseline: 29.0175x; 1.1834x over previous
import functools

import numpy as np
import jax
import jax.numpy as jnp
from jax.experimental import pallas as pl
from jax.experimental.pallas import tpu as pltpu

_N = 20000
_G = 64
_TOPK = 10
_SCALE_CLAMP = float(np.log(1000.0 / 16))


def _sim_ota_body(anchors_t, deltas_t, gt, cls2, stride2, out_ref):
    ax0 = anchors_t[0:1, :]
    ay0 = anchors_t[1:2, :]
    ax1 = anchors_t[2:3, :]
    ay1 = anchors_t[3:4, :]
    d0 = deltas_t[0:1, :]
    d1 = deltas_t[1:2, :]
    d2 = deltas_t[2:3, :]
    d3 = deltas_t[3:4, :]

    widths = ax1 - ax0
    heights = ay1 - ay0
    ctr_x = ax0 + 0.5 * widths
    ctr_y = ay0 + 0.5 * heights
    dx = d0 / 10.0
    dy = d1 / 10.0
    dw = jnp.minimum(d2 / 5.0, _SCALE_CLAMP)
    dh = jnp.minimum(d3 / 5.0, _SCALE_CLAMP)
    pcx = dx * widths + ctr_x
    pcy = dy * heights + ctr_y
    pw = jnp.exp(dw) * widths
    ph = jnp.exp(dh) * heights
    px0 = pcx - 0.5 * pw
    py0 = pcy - 0.5 * ph
    px1 = pcx + 0.5 * pw
    py1 = pcy + 0.5 * ph

    x_shifts = (ax0 + ax1) / 2.0
    y_shifts = (ay0 + ay1) / 2.0

    g0 = gt[:, 0:1]
    g1 = gt[:, 1:2]
    g2 = gt[:, 2:3]
    g3 = gt[:, 3:4]

    cdist = 1.5 * stride2[...]
    gt_cx = (g0 + g2) / 2.0
    gt_cy = (g1 + g3) / 2.0
    in_cx = jnp.abs(x_shifts - gt_cx) < cdist
    in_cy = jnp.abs(y_shifts - gt_cy) < cdist
    geom = in_cx & in_cy

    area_a = (g2 - g0) * (g3 - g1)
    area_p = (px1 - px0) * (py1 - py0)
    ltx = jnp.maximum(g0, px0)
    lty = jnp.maximum(g1, py0)
    rbx = jnp.minimum(g2, px1)
    rby = jnp.minimum(g3, py1)
    whx = jnp.clip(rbx - ltx, 0.0, None)
    why = jnp.clip(rby - lty, 0.0, None)
    inter = whx * why
    union = area_a + area_p - inter
    iou = inter / jnp.maximum(union, 1e-8)

    iou_loss = -jnp.log(iou + 1e-8)
    p = jax.nn.sigmoid(cls2[...])
    cls_loss = -jnp.log(p + 1e-12)
    cost = cls_loss + 3.0 * iou_loss + jnp.where(geom, 0.0, 1e6)

    col_ids = jax.lax.broadcasted_iota(jnp.int32, (_G, _N), 1)
    row_ids = jax.lax.broadcasted_iota(jnp.int32, (_G, _N), 0)

    iou_cand = jnp.where(geom, iou, 0.0)

    def _topk_iou(i, carry):
        s, work = carry
        m = jnp.max(work, axis=1, keepdims=True)
        amax = jnp.min(jnp.where(work == m, col_ids, _N), axis=1, keepdims=True)
        work = jnp.where(col_ids == amax, -jnp.inf, work)
        return s + m, work

    s0 = jnp.zeros((_G, 1), jnp.float32)
    s, _ = jax.lax.fori_loop(0, _TOPK, _topk_iou, (s0, iou_cand))
    dyn_k = jnp.maximum(s, 1.0).astype(jnp.int32)

    def _topk_cost(i, work):
        m = jnp.min(work, axis=1, keepdims=True)
        amin = jnp.min(jnp.where(work == m, col_ids, _N), axis=1, keepdims=True)
        hit = (col_ids == amin) & (dyn_k > i)
        return jnp.where(hit, jnp.inf, work)

    workc = jax.lax.fori_loop(0, _TOPK, _topk_cost, cost)
    matching = (workc != cost) & geom

    ones = jnp.where(matching, 1.0, 0.0)
    nmatch = jnp.sum(ones, axis=0, keepdims=True)
    multi = nmatch > 1.0
    cmin = jnp.min(cost, axis=0, keepdims=True)
    best_gt = jnp.min(jnp.where(cost == cmin, row_ids, _G), axis=0, keepdims=True)
    onehot = row_ids == best_gt
    matching = matching & ~(multi & ~onehot)
    matchf = jnp.where(matching, 1.0, 0.0)

    fg = jnp.sum(matchf, axis=0, keepdims=True) > 0.0
    pred_ious = jnp.sum(jnp.where(matching, iou, 0.0), axis=0, keepdims=True)

    area_b = (ax1 - ax0) * (ay1 - ay0)
    ltx2 = jnp.maximum(g0, ax0)
    lty2 = jnp.maximum(g1, ay0)
    rbx2 = jnp.minimum(g2, ax1)
    rby2 = jnp.minimum(g3, ay1)
    whx2 = jnp.clip(rbx2 - ltx2, 0.0, None)
    why2 = jnp.clip(rby2 - lty2, 0.0, None)
    inter2 = whx2 * why2
    union2 = area_a + area_b - inter2
    iou2 = inter2 / jnp.maximum(union2, 1e-8)
    max_anchor_iou = jnp.max(iou2, axis=0, keepdims=True)

    label = jnp.where(fg, 1.0,
                      jnp.where(max_anchor_iou >= 0.3, -1.0, 0.0))

    bx0 = jnp.sum(matchf * g0, axis=0, keepdims=True)
    bx1 = jnp.sum(matchf * g1, axis=0, keepdims=True)
    bx2 = jnp.sum(matchf * g2, axis=0, keepdims=True)
    bx3 = jnp.sum(matchf * g3, axis=0, keepdims=True)

    zero = jnp.zeros((1, _N), jnp.float32)
    out_ref[:, :] = jnp.concatenate(
        [label, pred_ious, bx0, bx1, bx2, bx3, zero, zero], axis=0)


@jax.jit
def kernel(anchors, pred_deltas, gt_boxes, cls_preds, expanded_strides, gt_classes):
    del gt_classes
    anchors_t = anchors.T
    deltas_t = pred_deltas.T
    cls2 = cls_preds.reshape(1, _N)
    stride2 = expanded_strides.reshape(1, _N)
    out = pl.pallas_call(
        _sim_ota_body,
        out_shape=jax.ShapeDtypeStruct((8, _N), jnp.float32),
    )(anchors_t, deltas_t, gt_boxes, cls2, stride2)
    return out[:6, :].T


# speedup vs baseline: 43.3119x; 1.4926x over previous
import functools

import numpy as np
import jax
import jax.numpy as jnp
from jax.experimental import pallas as pl
from jax.experimental.pallas import tpu as pltpu

_N = 20000
_G = 64
_TOPK = 10
_SCALE_CLAMP = float(np.log(1000.0 / 16))


def _sim_ota_body(anchors_t, deltas_t, gt, cls2, stride2, out_ref):
    ax0 = anchors_t[0:1, :]
    ay0 = anchors_t[1:2, :]
    ax1 = anchors_t[2:3, :]
    ay1 = anchors_t[3:4, :]
    d0 = deltas_t[0:1, :]
    d1 = deltas_t[1:2, :]
    d2 = deltas_t[2:3, :]
    d3 = deltas_t[3:4, :]

    widths = ax1 - ax0
    heights = ay1 - ay0
    ctr_x = ax0 + 0.5 * widths
    ctr_y = ay0 + 0.5 * heights
    dx = d0 / 10.0
    dy = d1 / 10.0
    dw = jnp.minimum(d2 / 5.0, _SCALE_CLAMP)
    dh = jnp.minimum(d3 / 5.0, _SCALE_CLAMP)
    pcx = dx * widths + ctr_x
    pcy = dy * heights + ctr_y
    pw = jnp.exp(dw) * widths
    ph = jnp.exp(dh) * heights
    px0 = pcx - 0.5 * pw
    py0 = pcy - 0.5 * ph
    px1 = pcx + 0.5 * pw
    py1 = pcy + 0.5 * ph

    x_shifts = (ax0 + ax1) / 2.0
    y_shifts = (ay0 + ay1) / 2.0

    g0 = gt[:, 0:1]
    g1 = gt[:, 1:2]
    g2 = gt[:, 2:3]
    g3 = gt[:, 3:4]

    cdist = 1.5 * stride2[...]
    gt_cx = (g0 + g2) / 2.0
    gt_cy = (g1 + g3) / 2.0
    in_cx = jnp.abs(x_shifts - gt_cx) < cdist
    in_cy = jnp.abs(y_shifts - gt_cy) < cdist
    geom = in_cx & in_cy

    area_a = (g2 - g0) * (g3 - g1)
    area_p = (px1 - px0) * (py1 - py0)
    ltx = jnp.maximum(g0, px0)
    lty = jnp.maximum(g1, py0)
    rbx = jnp.minimum(g2, px1)
    rby = jnp.minimum(g3, py1)
    whx = jnp.clip(rbx - ltx, 0.0, None)
    why = jnp.clip(rby - lty, 0.0, None)
    inter = whx * why
    union = area_a + area_p - inter
    iou = inter / jnp.maximum(union, 1e-8)

    iou_loss = -jnp.log(iou + 1e-8)
    p = jax.nn.sigmoid(cls2[...])
    cls_loss = -jnp.log(p + 1e-12)
    cost = cls_loss + 3.0 * iou_loss + jnp.where(geom, 0.0, 1e6)

    col_ids = jax.lax.broadcasted_iota(jnp.int32, (_G, _N), 1)
    row_ids = jax.lax.broadcasted_iota(jnp.int32, (_G, _N), 0)

    iou_cand = jnp.where(geom, iou, 0.0)

    iou_vals, iou_cnts = [], []
    cost_vals, cost_cnts = [], []
    mv_hi = jnp.full((_G, 1), jnp.inf)
    mv_lo = jnp.full((_G, 1), -jnp.inf)
    for _ in range(_TOPK):
        below = jnp.where(iou_cand < mv_hi, iou_cand, -jnp.inf)
        mv_hi = jnp.max(below, axis=1, keepdims=True)
        iou_vals.append(mv_hi)
        iou_cnts.append(jnp.sum(jnp.where(iou_cand == mv_hi, 1.0, 0.0),
                                axis=1, keepdims=True))
        above = jnp.where(cost > mv_lo, cost, jnp.inf)
        mv_lo = jnp.min(above, axis=1, keepdims=True)
        cost_vals.append(mv_lo)
        cost_cnts.append(jnp.sum(jnp.where(cost == mv_lo, 1.0, 0.0),
                                 axis=1, keepdims=True))

    s = jnp.zeros((_G, 1), jnp.float32)
    taken = jnp.zeros((_G, 1), jnp.float32)
    for i in range(_TOPK):
        take = jnp.minimum(iou_cnts[i], 10.0 - taken)
        s = s + jnp.where(take > 0.0, iou_vals[i] * take, 0.0)
        taken = taken + jnp.maximum(take, 0.0)
    dyn_k = jnp.maximum(s, 1.0).astype(jnp.int32)
    kf = dyn_k.astype(jnp.float32)

    t = jnp.full((_G, 1), jnp.inf)
    r = jnp.zeros((_G, 1), jnp.float32)
    c_star = jnp.ones((_G, 1), jnp.float32)
    cprev = jnp.zeros((_G, 1), jnp.float32)
    for i in range(_TOPK):
        hit = (cprev < kf) & (cprev + cost_cnts[i] >= kf)
        t = jnp.where(hit, cost_vals[i], t)
        r = jnp.where(hit, kf - cprev, r)
        c_star = jnp.where(hit, cost_cnts[i], c_star)
        cprev = cprev + cost_cnts[i]

    need_split = jnp.any((r < c_star) & (t < 1e5))

    def _tie_split(_):
        idx_cut = jnp.full((_G, 1), _N, jnp.int32)
        prev = jnp.full((_G, 1), -1, jnp.int32)
        ri = r.astype(jnp.int32)
        for j in range(1, _TOPK + 1):
            cand = jnp.where((cost == t) & (col_ids > prev), col_ids, _N)
            nxt = jnp.min(cand, axis=1, keepdims=True)
            idx_cut = jnp.where(ri == j, nxt, idx_cut)
            prev = nxt
        return jnp.where(r < c_star, idx_cut, _N)

    idx_cut = jax.lax.cond(need_split, _tie_split,
                           lambda _: jnp.full((_G, 1), _N, jnp.int32),
                           operand=None)

    matching = ((cost < t) | ((cost == t) & (col_ids <= idx_cut))) & geom

    ones = jnp.where(matching, 1.0, 0.0)
    nmatch = jnp.sum(ones, axis=0, keepdims=True)
    multi = nmatch > 1.0
    cmin = jnp.min(cost, axis=0, keepdims=True)
    best_gt = jnp.min(jnp.where(cost == cmin, row_ids, _G), axis=0, keepdims=True)
    onehot = row_ids == best_gt
    matching = matching & ~(multi & ~onehot)
    matchf = jnp.where(matching, 1.0, 0.0)

    fg = jnp.sum(matchf, axis=0, keepdims=True) > 0.0
    pred_ious = jnp.sum(jnp.where(matching, iou, 0.0), axis=0, keepdims=True)

    area_b = (ax1 - ax0) * (ay1 - ay0)
    ltx2 = jnp.maximum(g0, ax0)
    lty2 = jnp.maximum(g1, ay0)
    rbx2 = jnp.minimum(g2, ax1)
    rby2 = jnp.minimum(g3, ay1)
    whx2 = jnp.clip(rbx2 - ltx2, 0.0, None)
    why2 = jnp.clip(rby2 - lty2, 0.0, None)
    inter2 = whx2 * why2
    union2 = area_a + area_b - inter2
    iou2 = inter2 / jnp.maximum(union2, 1e-8)
    max_anchor_iou = jnp.max(iou2, axis=0, keepdims=True)

    label = jnp.where(fg, 1.0,
                      jnp.where(max_anchor_iou >= 0.3, -1.0, 0.0))

    bx0 = jnp.sum(matchf * g0, axis=0, keepdims=True)
    bx1 = jnp.sum(matchf * g1, axis=0, keepdims=True)
    bx2 = jnp.sum(matchf * g2, axis=0, keepdims=True)
    bx3 = jnp.sum(matchf * g3, axis=0, keepdims=True)

    zero = jnp.zeros((1, _N), jnp.float32)
    out_ref[:, :] = jnp.concatenate(
        [label, pred_ious, bx0, bx1, bx2, bx3, zero, zero], axis=0)


@jax.jit
def kernel(anchors, pred_deltas, gt_boxes, cls_preds, expanded_strides, gt_classes):
    del gt_classes
    anchors_t = anchors.T
    deltas_t = pred_deltas.T
    cls2 = cls_preds.reshape(1, _N)
    stride2 = expanded_strides.reshape(1, _N)
    out = pl.pallas_call(
        _sim_ota_body,
        out_shape=jax.ShapeDtypeStruct((8, _N), jnp.float32),
    )(anchors_t, deltas_t, gt_boxes, cls2, stride2)
    return out[:6, :].T
